# baseline (speedup 1.0000x reference)
import functools

import jax
import jax.numpy as jnp
import numpy as np
from jax import lax
from jax.experimental import pallas as pl
from jax.experimental.pallas import tpu as pltpu
from jax.experimental.pallas import tpu_sc as plsc

B, F, D = 16384, 26, 16
V = 2600000
N = B * F
_info = plsc.get_sparse_core_info()
NC, NS, L = _info.num_cores, _info.num_subcores, _info.num_lanes
NW = NC * NS

SLAB = 1024
NFULL = (V // 128) // 8
VREM0 = NFULL * SLAB
VREM = V - VREM0
APW = -(-NFULL // NW)

NPW = N // NW
PERIOD = (F * L) // np.gcd(F, L)
NSEG = PERIOD // L
NITER = NPW // PERIOD
CHUNK = 3328
NCHUNK = NPW // CHUNK

_FIELD_DIMS = [100000] * F
_OFFSETS = np.concatenate([[0], np.cumsum(_FIELD_DIMS)[:-1]]).astype(np.int32)
_PATTERN = _OFFSETS[np.arange(PERIOD) % F]


def _transpose_kernel(tt_hbm, trem_hbm, tlin_hbm, in0, in1, ob0, ob1,
                      rsem0, rsem1, wsem0, wsem1):
    wid = lax.axis_index("s") * NC + lax.axis_index("c")
    c0 = wid * APW
    end = jnp.minimum(c0 + APW, NFULL)
    iota = lax.iota(jnp.int32, L)

    ins = (in0, in1)
    obs = (ob0, ob1)
    rsems = (rsem0, rsem1)
    wsems = (wsem0, wsem1)

    def read(c, p):
        pltpu.async_copy(tt_hbm.at[:, pl.ds(c * SLAB, SLAB)], ins[p], rsems[p])

    def wait_read(p):
        pltpu.make_async_copy(tt_hbm.at[:, pl.ds(0, SLAB)], ins[p],
                              rsems[p]).wait()

    def write(c, p):
        pltpu.async_copy(obs[p], tlin_hbm.at[pl.ds(c * SLAB * D, SLAB * D)],
                         wsems[p])

    def wait_write(p):
        pltpu.make_async_copy(obs[p], tlin_hbm.at[pl.ds(0, SLAB * D)],
                              wsems[p]).wait()

    def transpose(p):
        inb, ob = ins[p], obs[p]
        def tj(j, c):
            base = j * L
            for u in range(L):
                col = base + u
                vals = plsc.load_gather(inb, [iota, jnp.full((L,), col,
                                                            jnp.int32)])
                ob[pl.ds(col * L, L)] = vals
            return c
        lax.fori_loop(0, SLAB // L, tj, 0)

    @pl.when(c0 < end)
    def _():
        read(c0, 0)

    def pair(k, carry):
        c = c0 + 2 * k

        @pl.when(c < end)
        def _():
            @pl.when(c + 1 < end)
            def _():
                read(c + 1, 1)
            wait_read(0)
            @pl.when(k > 0)
            def _():
                wait_write(0)
            transpose(0)
            write(c, 0)

        @pl.when(c + 1 < end)
        def _():
            @pl.when(c + 2 < end)
            def _():
                read(c + 2, 0)
            wait_read(1)
            @pl.when(k > 0)
            def _():
                wait_write(1)
            transpose(1)
            write(c + 1, 1)
        return carry

    lax.fori_loop(0, (APW + 1) // 2, pair, 0)

    nmine = jnp.maximum(end - c0, 0)
    @pl.when(nmine >= 1)
    def _():
        wait_write(0)
    @pl.when(nmine >= 2)
    def _():
        wait_write(1)

    @pl.when(wid == NW - 1)
    def _():
        pltpu.sync_copy(trem_hbm, ob0.at[pl.ds(0, VREM * D)])
        pltpu.sync_copy(ob0.at[pl.ds(0, VREM * D)],
                        tlin_hbm.at[pl.ds(VREM0 * D, VREM * D)])


@jax.jit
def _relayout(tt, trem):
    return pl.kernel(
        _transpose_kernel,
        out_type=jax.ShapeDtypeStruct((V * D,), jnp.float32),
        mesh=plsc.VectorSubcoreMesh(core_axis_name="c", subcore_axis_name="s"),
        scratch_types=[
            pltpu.VMEM((L, SLAB), jnp.float32),
            pltpu.VMEM((L, SLAB), jnp.float32),
            pltpu.VMEM((SLAB * D,), jnp.float32),
            pltpu.VMEM((SLAB * D,), jnp.float32),
            pltpu.SemaphoreType.DMA,
            pltpu.SemaphoreType.DMA,
            pltpu.SemaphoreType.DMA,
            pltpu.SemaphoreType.DMA,
        ],
        compiler_params=pltpu.CompilerParams(needs_layout_passes=False),
    )(tt, trem)


def _lookup_kernel(x_hbm, patt_hbm, t2d_hbm, out_hbm, idx_v, patt_v, rows_v,
                   sem):
    wid = lax.axis_index("s") * NC + lax.axis_index("c")
    base = wid * NPW

    pltpu.sync_copy(x_hbm.at[pl.ds(base, NPW)], idx_v)
    pltpu.sync_copy(patt_hbm, patt_v)

    pregs = [patt_v[pl.ds(u * L, L)] for u in range(NSEG)]

    def add_offsets(t, carry):
        s = t * PERIOD
        for u in range(NSEG):
            sl = pl.ds(s + u * L, L)
            idx_v[sl] = idx_v[sl] + pregs[u]
        return carry

    lax.fori_loop(0, NITER, add_offsets, 0)

    def do_chunk(k, carry):
        pltpu.async_copy(
            t2d_hbm.at[idx_v.at[pl.ds(k * CHUNK, CHUNK)]], rows_v, sem
        ).wait()
        pltpu.sync_copy(rows_v, out_hbm.at[pl.ds(base + k * CHUNK, CHUNK)])
        return carry

    lax.fori_loop(0, NCHUNK, do_chunk, 0)


@jax.jit
def _lookup(x_flat, patt, t2d):
    return pl.kernel(
        _lookup_kernel,
        out_type=jax.ShapeDtypeStruct((N, D), jnp.float32),
        mesh=plsc.VectorSubcoreMesh(core_axis_name="c", subcore_axis_name="s"),
        scratch_types=[
            pltpu.VMEM((NPW,), jnp.int32),
            pltpu.VMEM((PERIOD,), jnp.int32),
            pltpu.VMEM((CHUNK, D), jnp.float32),
            pltpu.SemaphoreType.DMA,
        ],
        compiler_params=pltpu.CompilerParams(use_tc_tiling_on_sc=False),
    )(x_flat, patt, t2d)


def kernel(x, table):
    patt = jnp.asarray(_PATTERN)
    trem = table[VREM0:].reshape(-1)
    tlin = _relayout(table.T, trem)
    t2d = tlin.reshape(V, D)
    out = _lookup(x.reshape(-1), patt, t2d)
    return out.reshape(B, F, D)


# speedup vs baseline: 1.8627x; 1.8627x over previous
import functools

import jax
import jax.numpy as jnp
import numpy as np
from jax import lax
from jax.experimental import pallas as pl
from jax.experimental.pallas import tpu as pltpu
from jax.experimental.pallas import tpu_sc as plsc

B, F, D = 16384, 26, 16
V = 2600000
N = B * F
_info = plsc.get_sparse_core_info()
NC, NS, L = _info.num_cores, _info.num_subcores, _info.num_lanes
NW = NC * NS

SLAB = 1024
NFULL = (V // 128) // 8
VREM0 = NFULL * SLAB
VREM = V - VREM0
APW = -(-NFULL // NW)

NPW = N // NW
PERIOD = (F * L) // np.gcd(F, L)
NSEG = PERIOD // L
NITER = NPW // PERIOD
CHUNK = 3328
NCHUNK = NPW // CHUNK

_FIELD_DIMS = [100000] * F
_OFFSETS = np.concatenate([[0], np.cumsum(_FIELD_DIMS)[:-1]]).astype(np.int32)
_PATTERN = _OFFSETS[np.arange(PERIOD) % F]


def _transpose_kernel(tt_hbm, trem_hbm, tlin_hbm, in0, in1, ob0, ob1,
                      rsem0, rsem1, wsem0, wsem1):
    wid = lax.axis_index("s") * NC + lax.axis_index("c")
    c0 = wid * APW
    end = jnp.minimum(c0 + APW, NFULL)
    iota = lax.iota(jnp.int32, L)

    ins = (in0, in1)
    obs = (ob0, ob1)
    rsems = (rsem0, rsem1)
    wsems = (wsem0, wsem1)

    def read(c, p):
        pltpu.async_copy(tt_hbm.at[:, pl.ds(c * SLAB, SLAB)], ins[p], rsems[p])

    def wait_read(p):
        pltpu.make_async_copy(tt_hbm.at[:, pl.ds(0, SLAB)], ins[p],
                              rsems[p]).wait()

    def write(c, p):
        pltpu.async_copy(obs[p], tlin_hbm.at[pl.ds(c * SLAB * D, SLAB * D)],
                         wsems[p])

    def wait_write(p):
        pltpu.make_async_copy(obs[p], tlin_hbm.at[pl.ds(0, SLAB * D)],
                              wsems[p]).wait()

    def transpose(p):
        inb, ob = ins[p], obs[p]
        def tj(j, c):
            jbase = iota * D + j * L * D
            for d in range(D):
                row = inb[d, pl.ds(j * L, L)]
                plsc.store_scatter(ob, [jbase + d], row)
            return c
        lax.fori_loop(0, SLAB // L, tj, 0)

    @pl.when(c0 < end)
    def _():
        read(c0, 0)

    def pair(k, carry):
        c = c0 + 2 * k

        @pl.when(c < end)
        def _():
            @pl.when(c + 1 < end)
            def _():
                read(c + 1, 1)
            wait_read(0)
            @pl.when(k > 0)
            def _():
                wait_write(0)
            transpose(0)
            write(c, 0)

        @pl.when(c + 1 < end)
        def _():
            @pl.when(c + 2 < end)
            def _():
                read(c + 2, 0)
            wait_read(1)
            @pl.when(k > 0)
            def _():
                wait_write(1)
            transpose(1)
            write(c + 1, 1)
        return carry

    lax.fori_loop(0, (APW + 1) // 2, pair, 0)

    nmine = jnp.maximum(end - c0, 0)
    @pl.when(nmine >= 1)
    def _():
        wait_write(0)
    @pl.when(nmine >= 2)
    def _():
        wait_write(1)

    @pl.when(wid == NW - 1)
    def _():
        pltpu.sync_copy(trem_hbm, ob0.at[pl.ds(0, VREM * D)])
        pltpu.sync_copy(ob0.at[pl.ds(0, VREM * D)],
                        tlin_hbm.at[pl.ds(VREM0 * D, VREM * D)])


@jax.jit
def _relayout(tt, trem):
    return pl.kernel(
        _transpose_kernel,
        out_type=jax.ShapeDtypeStruct((V * D,), jnp.float32),
        mesh=plsc.VectorSubcoreMesh(core_axis_name="c", subcore_axis_name="s"),
        scratch_types=[
            pltpu.VMEM((L, SLAB), jnp.float32),
            pltpu.VMEM((L, SLAB), jnp.float32),
            pltpu.VMEM((SLAB * D,), jnp.float32),
            pltpu.VMEM((SLAB * D,), jnp.float32),
            pltpu.SemaphoreType.DMA,
            pltpu.SemaphoreType.DMA,
            pltpu.SemaphoreType.DMA,
            pltpu.SemaphoreType.DMA,
        ],
        compiler_params=pltpu.CompilerParams(needs_layout_passes=False),
    )(tt, trem)


def _lookup_kernel(x_hbm, patt_hbm, t2d_hbm, out_hbm, idx_v, patt_v, rows_v,
                   sem):
    wid = lax.axis_index("s") * NC + lax.axis_index("c")
    base = wid * NPW

    pltpu.sync_copy(x_hbm.at[pl.ds(base, NPW)], idx_v)
    pltpu.sync_copy(patt_hbm, patt_v)

    pregs = [patt_v[pl.ds(u * L, L)] for u in range(NSEG)]

    def add_offsets(t, carry):
        s = t * PERIOD
        for u in range(NSEG):
            sl = pl.ds(s + u * L, L)
            idx_v[sl] = idx_v[sl] + pregs[u]
        return carry

    lax.fori_loop(0, NITER, add_offsets, 0)

    def do_chunk(k, carry):
        pltpu.async_copy(
            t2d_hbm.at[idx_v.at[pl.ds(k * CHUNK, CHUNK)]], rows_v, sem
        ).wait()
        pltpu.sync_copy(rows_v, out_hbm.at[pl.ds(base + k * CHUNK, CHUNK)])
        return carry

    lax.fori_loop(0, NCHUNK, do_chunk, 0)


@jax.jit
def _lookup(x_flat, patt, t2d):
    return pl.kernel(
        _lookup_kernel,
        out_type=jax.ShapeDtypeStruct((N, D), jnp.float32),
        mesh=plsc.VectorSubcoreMesh(core_axis_name="c", subcore_axis_name="s"),
        scratch_types=[
            pltpu.VMEM((NPW,), jnp.int32),
            pltpu.VMEM((PERIOD,), jnp.int32),
            pltpu.VMEM((CHUNK, D), jnp.float32),
            pltpu.SemaphoreType.DMA,
        ],
        compiler_params=pltpu.CompilerParams(use_tc_tiling_on_sc=False),
    )(x_flat, patt, t2d)


def kernel(x, table):
    patt = jnp.asarray(_PATTERN)
    trem = table[VREM0:].reshape(-1)
    tlin = _relayout(table.T, trem)
    t2d = tlin.reshape(V, D)
    out = _lookup(x.reshape(-1), patt, t2d)
    return out.reshape(B, F, D)


# speedup vs baseline: 2.3591x; 1.2665x over previous
import functools

import jax
import jax.numpy as jnp
import numpy as np
from jax import lax
from jax.experimental import pallas as pl
from jax.experimental.pallas import tpu as pltpu
from jax.experimental.pallas import tpu_sc as plsc

B, F, D = 16384, 26, 16
V = 2600000
N = B * F
_info = plsc.get_sparse_core_info()
NC, NS, L = _info.num_cores, _info.num_subcores, _info.num_lanes
NW = NC * NS

SLAB = 1024
NFULL = (V // 128) // 8
VREM0 = NFULL * SLAB
VREM = V - VREM0
APW = -(-NFULL // NW)

NPW = N // NW
PERIOD = (F * L) // np.gcd(F, L)
NSEG = PERIOD // L
NITER = NPW // PERIOD
CHUNK = 3328
NCHUNK = NPW // CHUNK

_FIELD_DIMS = [100000] * F
_OFFSETS = np.concatenate([[0], np.cumsum(_FIELD_DIMS)[:-1]]).astype(np.int32)
_PATTERN = _OFFSETS[np.arange(PERIOD) % F]


def _transpose_kernel(tt_hbm, trem_hbm, tlin_hbm, in0, in1, ob0, ob1,
                      rsem0, rsem1, wsem0, wsem1):
    wid = lax.axis_index("s") * NC + lax.axis_index("c")
    c0 = wid * APW
    end = jnp.minimum(c0 + APW, NFULL)
    iota = lax.iota(jnp.int32, L)

    ins = (in0, in1)
    obs = (ob0, ob1)
    rsems = (rsem0, rsem1)
    wsems = (wsem0, wsem1)

    def read(c, p):
        pltpu.async_copy(tt_hbm.at[:, pl.ds(c * SLAB, SLAB)], ins[p], rsems[p])

    def wait_read(p):
        pltpu.make_async_copy(tt_hbm.at[:, pl.ds(0, SLAB)], ins[p],
                              rsems[p]).wait()

    def write(c, p):
        pltpu.async_copy(obs[p], tlin_hbm.at[pl.ds(c * SLAB * D, SLAB * D)],
                         wsems[p])

    def wait_write(p):
        pltpu.make_async_copy(obs[p], tlin_hbm.at[pl.ds(0, SLAB * D)],
                              wsems[p]).wait()

    def transpose(p):
        inb, ob = ins[p], obs[p]
        @functools.partial(plsc.parallel_loop, 0, SLAB // L, unroll=4)
        def _(j):
            jbase = iota * D + j * (L * D)
            for d in range(D):
                row = inb[d, pl.ds(j * L, L)]
                plsc.store_scatter(ob, [jbase + d], row)

    @pl.when(c0 < end)
    def _():
        read(c0, 0)

    def pair(k, carry):
        c = c0 + 2 * k

        @pl.when(c < end)
        def _():
            @pl.when(c + 1 < end)
            def _():
                read(c + 1, 1)
            wait_read(0)
            @pl.when(k > 0)
            def _():
                wait_write(0)
            transpose(0)
            write(c, 0)

        @pl.when(c + 1 < end)
        def _():
            @pl.when(c + 2 < end)
            def _():
                read(c + 2, 0)
            wait_read(1)
            @pl.when(k > 0)
            def _():
                wait_write(1)
            transpose(1)
            write(c + 1, 1)
        return carry

    lax.fori_loop(0, (APW + 1) // 2, pair, 0)

    nmine = jnp.maximum(end - c0, 0)
    @pl.when(nmine >= 1)
    def _():
        wait_write(0)
    @pl.when(nmine >= 2)
    def _():
        wait_write(1)

    @pl.when(wid == NW - 1)
    def _():
        pltpu.sync_copy(trem_hbm, ob0.at[pl.ds(0, VREM * D)])
        pltpu.sync_copy(ob0.at[pl.ds(0, VREM * D)],
                        tlin_hbm.at[pl.ds(VREM0 * D, VREM * D)])


@jax.jit
def _relayout(tt, trem):
    return pl.kernel(
        _transpose_kernel,
        out_type=jax.ShapeDtypeStruct((V * D,), jnp.float32),
        mesh=plsc.VectorSubcoreMesh(core_axis_name="c", subcore_axis_name="s"),
        scratch_types=[
            pltpu.VMEM((L, SLAB), jnp.float32),
            pltpu.VMEM((L, SLAB), jnp.float32),
            pltpu.VMEM((SLAB * D,), jnp.float32),
            pltpu.VMEM((SLAB * D,), jnp.float32),
            pltpu.SemaphoreType.DMA,
            pltpu.SemaphoreType.DMA,
            pltpu.SemaphoreType.DMA,
            pltpu.SemaphoreType.DMA,
        ],
        compiler_params=pltpu.CompilerParams(needs_layout_passes=False),
    )(tt, trem)


def _lookup_kernel(x_hbm, patt_hbm, t2d_hbm, out_hbm, idx_v, patt_v, rows_v,
                   sem):
    wid = lax.axis_index("s") * NC + lax.axis_index("c")
    base = wid * NPW

    pltpu.sync_copy(x_hbm.at[pl.ds(base, NPW)], idx_v)
    pltpu.sync_copy(patt_hbm, patt_v)

    pregs = [patt_v[pl.ds(u * L, L)] for u in range(NSEG)]

    def add_offsets(t, carry):
        s = t * PERIOD
        for u in range(NSEG):
            sl = pl.ds(s + u * L, L)
            idx_v[sl] = idx_v[sl] + pregs[u]
        return carry

    lax.fori_loop(0, NITER, add_offsets, 0)

    def do_chunk(k, carry):
        pltpu.async_copy(
            t2d_hbm.at[idx_v.at[pl.ds(k * CHUNK, CHUNK)]], rows_v, sem
        ).wait()
        pltpu.sync_copy(rows_v, out_hbm.at[pl.ds(base + k * CHUNK, CHUNK)])
        return carry

    lax.fori_loop(0, NCHUNK, do_chunk, 0)


@jax.jit
def _lookup(x_flat, patt, t2d):
    return pl.kernel(
        _lookup_kernel,
        out_type=jax.ShapeDtypeStruct((N, D), jnp.float32),
        mesh=plsc.VectorSubcoreMesh(core_axis_name="c", subcore_axis_name="s"),
        scratch_types=[
            pltpu.VMEM((NPW,), jnp.int32),
            pltpu.VMEM((PERIOD,), jnp.int32),
            pltpu.VMEM((CHUNK, D), jnp.float32),
            pltpu.SemaphoreType.DMA,
        ],
        compiler_params=pltpu.CompilerParams(use_tc_tiling_on_sc=False),
    )(x_flat, patt, t2d)


def kernel(x, table):
    patt = jnp.asarray(_PATTERN)
    trem = table[VREM0:].reshape(-1)
    tlin = _relayout(table.T, trem)
    t2d = tlin.reshape(V, D)
    out = _lookup(x.reshape(-1), patt, t2d)
    return out.reshape(B, F, D)
